# Initial kernel scaffold; baseline (speedup 1.0000x reference)
import jax
import jax.numpy as jnp
from jax.experimental import pallas as pl


def kernel(contours, scores, boxes, uncertainties):
    raise NotImplementedError("write your pallas kernel here")


# speedup vs baseline: 23.8819x; 23.8819x over previous
import jax
import jax.numpy as jnp
from jax.experimental import pallas as pl
from jax.experimental.pallas import tpu as pltpu

_N = 5000
_NP = 5120
_T = 128
_THR = 0.5


def _nms_kernel(boxes_ref, boxesT_ref, scoresT_ref, uncT_ref,
                scores_ref, unc_ref, cont_ref, out_ref,
                wc_ref, keepc_ref, votesc_ref):
    x1r = boxesT_ref[0:1, :]
    y1r = boxesT_ref[1:2, :]
    x2r = boxesT_ref[2:3, :]
    y2r = boxesT_ref[3:4, :]
    area_r = (x2r - x1r) * (y2r - y1r)
    su_r = (uncT_ref[0:1, :] + uncT_ref[1:2, :]
            + uncT_ref[2:3, :] + uncT_ref[3:4, :])
    w_r = scoresT_ref[0:1, :] * (1.0 - jax.nn.sigmoid(su_r * 0.25))
    idx_r = jax.lax.broadcasted_iota(jnp.int32, (1, _NP), 1)

    unc = unc_ref[...]
    su_c = unc[:, 0:1] + unc[:, 1:2] + unc[:, 2:3] + unc[:, 3:4]
    wc_ref[...] = scores_ref[...] * (1.0 - jax.nn.sigmoid(su_c * 0.25))
    keepc_ref[...] = jnp.ones((_NP, 1), jnp.float32)

    ntiles = _NP // _T

    def tile_body(t, carry):
        votes_r, keep_r = carry
        r0 = t * _T
        sl = pl.ds(r0, _T)
        bx = boxes_ref[sl, :]
        x1c = bx[:, 0:1]
        y1c = bx[:, 1:2]
        x2c = bx[:, 2:3]
        y2c = bx[:, 3:4]
        ltx = jnp.maximum(x1c, x1r)
        lty = jnp.maximum(y1c, y1r)
        rbx = jnp.minimum(x2c, x2r)
        rby = jnp.minimum(y2c, y2r)
        inter = jnp.maximum(rbx - ltx, 0.0) * jnp.maximum(rby - lty, 0.0)
        union = (x2c - x1c) * (y2c - y1c) + area_r - inter
        iou = inter / jnp.maximum(union, 1e-9)
        ov = iou > _THR
        wc_t = wc_ref[sl, :]
        ic_t = jax.lax.broadcasted_iota(jnp.int32, (_T, 1), 0) + r0
        pri = (wc_t > w_r) | ((wc_t == w_r) & (ic_t < idx_r))
        m = ov & pri
        mT = ov & jnp.logical_not(pri) & (ic_t != idx_r)
        kc_t = keepc_ref[sl, :]
        votes_r = votes_r + jnp.sum(
            jnp.where(m, kc_t, 0.0), axis=0, keepdims=True)
        votesc_ref[sl, :] = jnp.sum(
            jnp.where(mT, keep_r, 0.0), axis=1, keepdims=True)
        return votes_r, keep_r

    def body(state):
        _, _, keep_r = state
        votes_r, _ = jax.lax.fori_loop(
            0, ntiles, tile_body, (jnp.zeros((1, _NP), jnp.float32), keep_r))
        new_r = (votes_r == 0.0).astype(jnp.float32)
        keepc_ref[...] = (votesc_ref[...] == 0.0).astype(jnp.float32)
        changed = jnp.any(new_r != keep_r)
        return (state[0] + 1, changed, new_r)

    def cond(state):
        return jnp.logical_and(state[1], state[0] < _NP)

    init = (jnp.int32(0), jnp.bool_(True), jnp.ones((1, _NP), jnp.float32))
    jax.lax.while_loop(cond, body, init)

    out = jnp.concatenate(
        [boxes_ref[...], scores_ref[...], unc, wc_ref[...], cont_ref[...]],
        axis=1)
    out_ref[...] = out * keepc_ref[...]


def kernel(contours, scores, boxes, uncertainties):
    pad = _NP - _N
    boxes_p = jnp.pad(boxes, ((0, pad), (0, 0)))
    scores_p = jnp.pad(scores, (0, pad))[:, None]
    unc_p = jnp.pad(uncertainties, ((0, pad), (0, 0)))
    cont_p = jnp.pad(contours.reshape(_N, -1), ((0, pad), (0, 0)))

    out = pl.pallas_call(
        _nms_kernel,
        out_shape=jax.ShapeDtypeStruct((_NP, 74), jnp.float32),
        scratch_shapes=[
            pltpu.VMEM((_NP, 1), jnp.float32),
            pltpu.VMEM((_NP, 1), jnp.float32),
            pltpu.VMEM((_NP, 1), jnp.float32),
        ],
    )(boxes_p, boxes_p.T, scores_p.T, unc_p.T, scores_p, unc_p, cont_p)
    return out[:_N]
